# Initial kernel scaffold; baseline (speedup 1.0000x reference)
import jax
import jax.numpy as jnp
from jax.experimental import pallas as pl


def kernel(output, mask, ind, target):
    raise NotImplementedError("write your pallas kernel here")


# speedup vs baseline: 1.3693x; 1.3693x over previous
import functools

import jax
import jax.numpy as jnp
from jax import lax
from jax.experimental import pallas as pl
from jax.experimental.pallas import tpu as pltpu
from jax.experimental.pallas import tpu_sc as plsc

B, D, H, W, M = 4, 8, 512, 512, 500
HW = H * W
NI = B * M
PER_TILE = 128
NT = 16


def _body(out_hbm, ind_hbm, mask_hbm, tgt_hbm, part_hbm, res_hbm,
          ind_v, mask_v, maskf_v, base_v, tgt_v, idx2, pred2,
          accb, part_v, all_v, sem):
    w = lax.axis_index("s")
    start = pl.multiple_of(jnp.minimum(w * PER_TILE, NI - PER_TILE), 8)
    thresh = w * PER_TILE - start

    c1 = pltpu.async_copy(ind_hbm.at[pl.ds(start, PER_TILE)], ind_v, sem)
    c2 = pltpu.async_copy(mask_hbm.at[pl.ds(start, PER_TILE)], mask_v, sem)
    c3 = pltpu.async_copy(tgt_hbm.at[pl.ds(start * D, PER_TILE * D)], tgt_v,
                          sem)
    c1.wait()
    c2.wait()
    c3.wait()

    iota = lax.iota(jnp.int32, 16)
    half = iota // 8
    doff = (iota % 8) * HW
    start_v = jnp.full((16,), start, jnp.int32)
    thresh_v = jnp.full((16,), thresh, jnp.int32)

    for g in range(8):
        item = start_v + (g * 16 + iota)
        b = ((item >= M).astype(jnp.int32) + (item >= 2 * M).astype(jnp.int32)
             + (item >= 3 * M).astype(jnp.int32))
        base_v[pl.ds(g * 16, 16)] = b * (D * HW) + ind_v[pl.ds(g * 16, 16)]
        valid = (g * 16 + iota) >= thresh_v
        maskf_v[pl.ds(g * 16, 16)] = (
            mask_v[pl.ds(g * 16, 16)].astype(jnp.float32)
            * valid.astype(jnp.float32))

    for v in range(64):
        bse = plsc.load_gather(base_v, [v * 2 + half])
        idx2[v // 8, pl.ds((v % 8) * 16, 16)] = bse + doff

    gathers = [
        pltpu.async_copy(out_hbm.at[idx2.at[k]], pred2.at[k], sem)
        for k in range(8)
    ]
    for g in gathers:
        g.wait()

    numacc = jnp.zeros((16,), jnp.float32)
    for g in range(8):
        numacc = numacc + maskf_v[pl.ds(g * 16, 16)]

    acc = jnp.zeros((16,), jnp.float32)
    for v in range(64):
        p = pred2[v // 8, pl.ds((v % 8) * 16, 16)]
        t = tgt_v[pl.ds(v * 16, 16)]
        mf = plsc.load_gather(maskf_v, [v * 2 + half])
        acc = acc + mf * jnp.abs(p - t)

    accb[...] = acc
    lossv = acc + plsc.load_gather(accb, [(iota + 8) & 15])
    num_v = jnp.full((16,), jnp.sum(numacc), jnp.float32)
    zero_v = jnp.zeros((16,), jnp.float32)
    part_v[...] = jnp.where(iota < 8, lossv,
                            jnp.where(iota == 8, num_v, zero_v))
    pltpu.sync_copy(part_v, part_hbm.at[w])
    plsc.subcore_barrier()
    plsc.subcore_barrier()

    @pl.when(w == 0)
    def _():
        pltpu.async_copy(part_hbm, all_v, sem).wait()
        tot = jnp.zeros((16,), jnp.float32)
        for si in range(NT):
            tot = tot + all_v[si, pl.ds(0, 16)]
        accb[...] = tot
        numv = plsc.load_gather(accb, [jnp.full((16,), 8, jnp.int32)])
        part_v[...] = tot / jnp.maximum(numv,
                                        jnp.full((16,), 1.0, jnp.float32))
        pltpu.sync_copy(part_v.at[pl.ds(0, 8)], res_hbm)


@jax.jit
def _run(outflat, indflat, maskflat, tgtflat):
    mesh = plsc.VectorSubcoreMesh(
        core_axis_name="c", subcore_axis_name="s", num_cores=1)
    _, res = pl.kernel(
        _body,
        out_type=(jax.ShapeDtypeStruct((NT, 16), jnp.float32),
                  jax.ShapeDtypeStruct((D,), jnp.float32)),
        mesh=mesh,
        compiler_params=pltpu.CompilerParams(needs_layout_passes=False),
        scratch_types=[
            pltpu.VMEM((PER_TILE,), jnp.int32),
            pltpu.VMEM((PER_TILE,), jnp.int32),
            pltpu.VMEM((PER_TILE,), jnp.float32),
            pltpu.VMEM((PER_TILE,), jnp.int32),
            pltpu.VMEM((PER_TILE * D,), jnp.float32),
            pltpu.VMEM((8, PER_TILE), jnp.int32),
            pltpu.VMEM((8, PER_TILE), jnp.float32),
            pltpu.VMEM((16,), jnp.float32),
            pltpu.VMEM((16,), jnp.float32),
            pltpu.VMEM((NT, 16), jnp.float32),
            pltpu.SemaphoreType.DMA,
        ],
    )(outflat, indflat, maskflat, tgtflat)
    return res


def kernel(output, mask, ind, target):
    return _run(output.reshape(-1), ind.reshape(-1), mask.reshape(-1),
                target.reshape(-1))


# speedup vs baseline: 1.7662x; 1.2898x over previous
import functools

import jax
import jax.numpy as jnp
from jax import lax
from jax.experimental import pallas as pl
from jax.experimental.pallas import tpu as pltpu
from jax.experimental.pallas import tpu_sc as plsc

B, D, H, W, M = 4, 8, 512, 512, 500
MP = 512
NIP = B * MP
PER_TILE = 128
DPC = D // 2
CHUNK = 64
NCH = DPC * (PER_TILE // CHUNK)


def _body(out2_hbm, ind_hbm, mask_hbm, tgt_hbm, part_hbm, res0_hbm, res1_hbm,
          ind_v, mask_v, maskf_v, w_v, r0, r1, r2, r3, r4, r5, r6, r7,
          tgt_v, rb0, rb1, accb, part_v, all_v, sem, semA, semB):
    rows_bufs = (r0, r1, r2, r3, r4, r5, r6, r7)
    c = lax.axis_index("c")
    s = lax.axis_index("s")
    b = s >> 2
    sub = s & 3
    ib = pl.multiple_of(b * MP + sub * PER_TILE, 8)
    base_d = c * DPC

    c1 = pltpu.async_copy(ind_hbm.at[pl.ds(ib, PER_TILE)], ind_v, sem)
    c2 = pltpu.async_copy(mask_hbm.at[pl.ds(ib, PER_TILE)], mask_v, sem)
    c3 = pltpu.async_copy(tgt_hbm.at[pl.ds(ib * D, PER_TILE * D)], tgt_v,
                          sem)
    c1.wait()
    c2.wait()
    c3.wait()

    iota = lax.iota(jnp.int32, 16)
    pbase_v = jnp.full((16,), (b * D + base_d) * H, jnp.int32)
    for g in range(8):
        iv = ind_v[pl.ds(g * 16, 16)]
        w_v[pl.ds(g * 16, 16)] = iv & 511
        hrow = pbase_v + lax.shift_right_logical(iv, 9)
        for dd in range(DPC):
            k = dd * 2 + (g >> 2)
            col = (g & 3) * 16
            rows_bufs[k][pl.ds(col, 16)] = hrow + dd * H
    for g in range(8):
        maskf_v[pl.ds(g * 16, 16)] = mask_v[pl.ds(g * 16, 16)].astype(
            jnp.float32)

    rbufs = (rb0, rb1)
    sems = (semA, semB)

    def fire(k):
        return pltpu.async_copy(out2_hbm.at[rows_bufs[k]], rbufs[k % 2],
                                sems[k % 2])

    copies = [None] * NCH
    copies[0] = fire(0)
    copies[1] = fire(1)

    accs = [jnp.zeros((16,), jnp.float32) for _ in range(DPC)]
    for k in range(NCH):
        dd = k // 2
        ch = k % 2
        copies[k].wait()
        rb = rbufs[k % 2]
        acc = accs[dd]
        for g in range(4):
            i0 = ch * CHUNK + g * 16
            col = w_v[pl.ds(i0, 16)]
            p = plsc.load_gather(rb, [g * 16 + iota, col])
            t = plsc.load_gather(tgt_v, [(i0 + iota) * D + (base_d + dd)])
            m = maskf_v[pl.ds(i0, 16)]
            acc = acc + m * jnp.abs(p - t)
        accs[dd] = acc
        if k + 2 < NCH:
            copies[k + 2] = fire(k + 2)

    numacc = jnp.zeros((16,), jnp.float32)
    for g in range(8):
        numacc = numacc + maskf_v[pl.ds(g * 16, 16)]

    part = jnp.zeros((16,), jnp.float32)
    for dd in range(DPC):
        dsum = jnp.full((16,), jnp.sum(accs[dd]), jnp.float32)
        dlane = jnp.full((16,), base_d + dd, jnp.int32)
        part = jnp.where(iota == dlane, dsum, part)
    num_v = jnp.full((16,), jnp.sum(numacc), jnp.float32)
    part = jnp.where(iota == 8, num_v, part)
    part_v[...] = part
    pltpu.sync_copy(part_v, part_hbm.at[c * 16 + s])
    plsc.subcore_barrier()
    plsc.subcore_barrier()

    @pl.when(s == 0)
    def _():
        pltpu.async_copy(part_hbm.at[pl.ds(c * 16, 16)], all_v, sem).wait()
        tot = jnp.zeros((16,), jnp.float32)
        for si in range(16):
            tot = tot + all_v[si, pl.ds(0, 16)]
        accb[...] = tot
        numv = plsc.load_gather(accb, [jnp.full((16,), 8, jnp.int32)])
        part_v[...] = tot / jnp.maximum(numv,
                                        jnp.full((16,), 1.0, jnp.float32))

        @pl.when(c == 0)
        def _():
            pltpu.sync_copy(part_v.at[pl.ds(0, 8)], res0_hbm)

        @pl.when(c == 1)
        def _():
            pltpu.sync_copy(part_v.at[pl.ds(0, 8)], res1_hbm)


@jax.jit
def _run(out2, indflat, maskflat, tgtflat):
    mesh = plsc.VectorSubcoreMesh(core_axis_name="c", subcore_axis_name="s")
    _, res0, res1 = pl.kernel(
        _body,
        out_type=(jax.ShapeDtypeStruct((32, 16), jnp.float32),
                  jax.ShapeDtypeStruct((D,), jnp.float32),
                  jax.ShapeDtypeStruct((D,), jnp.float32)),
        mesh=mesh,
        compiler_params=pltpu.CompilerParams(
            needs_layout_passes=False, use_tc_tiling_on_sc=True),
        scratch_types=[
            pltpu.VMEM((PER_TILE,), jnp.int32),
            pltpu.VMEM((PER_TILE,), jnp.int32),
            pltpu.VMEM((PER_TILE,), jnp.float32),
            pltpu.VMEM((PER_TILE,), jnp.int32),
            pltpu.VMEM((CHUNK,), jnp.int32),
            pltpu.VMEM((CHUNK,), jnp.int32),
            pltpu.VMEM((CHUNK,), jnp.int32),
            pltpu.VMEM((CHUNK,), jnp.int32),
            pltpu.VMEM((CHUNK,), jnp.int32),
            pltpu.VMEM((CHUNK,), jnp.int32),
            pltpu.VMEM((CHUNK,), jnp.int32),
            pltpu.VMEM((CHUNK,), jnp.int32),
            pltpu.VMEM((PER_TILE * D,), jnp.float32),
            pltpu.VMEM((CHUNK, W), jnp.float32),
            pltpu.VMEM((CHUNK, W), jnp.float32),
            pltpu.VMEM((16,), jnp.float32),
            pltpu.VMEM((16,), jnp.float32),
            pltpu.VMEM((16, 16), jnp.float32),
            pltpu.SemaphoreType.DMA,
            pltpu.SemaphoreType.DMA,
            pltpu.SemaphoreType.DMA,
        ],
    )(out2, indflat, maskflat, tgtflat)
    return res0 + res1


def kernel(output, mask, ind, target):
    pad = ((0, 0), (0, MP - M))
    ind_p = jnp.pad(ind, pad).reshape(-1)
    mask_p = jnp.pad(mask, pad).reshape(-1)
    tgt_p = jnp.pad(target, ((0, 0), (0, MP - M), (0, 0))).reshape(-1)
    return _run(output.reshape(B * D * H, W), ind_p, mask_p, tgt_p)
